# baseline (speedup 1.0000x reference)
import functools

import jax
import jax.numpy as jnp
from jax import lax
from jax.experimental import pallas as pl
from jax.experimental.pallas import tpu as pltpu
from jax.experimental.pallas import tpu_sc as plsc

_NC = 2
_NS = 16
_NW = _NC * _NS
_CH = 128
_BN = 256


def _cdiv(a, b):
    return (a + b - 1) // b


def _sc_scatter_partials(y, src, dst, zerosH):
    N, H = y.shape
    _, _, NHALF, CH = dst.shape
    RPT = N // _NS
    mesh = plsc.VectorSubcoreMesh(core_axis_name="c", subcore_axis_name="s")

    @functools.partial(
        pl.kernel,
        out_type=jax.ShapeDtypeStruct((_NC, N, H), jnp.float32),
        mesh=mesh,
        scratch_types=[
            pltpu.VMEM((NHALF, CH), jnp.int32),
            pltpu.VMEM((NHALF, CH), jnp.int32),
        ]
        + [pltpu.VMEM((CH, H), jnp.float32)] * 2
        + [pltpu.VMEM_SHARED((N, H), jnp.float32)]
        + [pltpu.SemaphoreType.DMA] * 2,
    )
    def k(y_hbm, src_hbm, dst_hbm, zero_hbm, out_hbm, src_half, dst_half,
          rb0, rb1, acc_sh, g0, g1):
        rows = (rb0, rb1)
        gs = (g0, g1)
        c = lax.axis_index("c")
        s = lax.axis_index("s")
        wid = c * _NS + s
        r0 = s * RPT

        @pl.when(c == 0)
        def _():
            pltpu.sync_copy(y_hbm.at[pl.ds(r0, RPT)], acc_sh.at[pl.ds(r0, RPT)])

        @pl.when(c != 0)
        def _():
            pltpu.sync_copy(zero_hbm.at[pl.ds(r0, RPT)], acc_sh.at[pl.ds(r0, RPT)])

        plsc.subcore_barrier()

        def fire_g(kk, b):
            pltpu.async_copy(y_hbm.at[src_half.at[kk]], rows[b], gs[b])

        def wait_g(b):
            pltpu.make_async_copy(
                y_hbm.at[src_half.at[0]], rows[b], gs[b]).wait()

        for half in range(2):
            pltpu.sync_copy(src_hbm.at[wid, half], src_half)
            pltpu.sync_copy(dst_hbm.at[wid, half], dst_half)

            def step(kk, b, fire_next):
                wait_g(b)
                if fire_next:
                    fire_g(kk + 1, 1 - b)
                pltpu.sync_copy(rows[b], acc_sh.at[dst_half.at[kk]], add=True)

            fire_g(0, 0)
            step(0, 0, True)
            step(1, 1, True)

            def body(r, carry):
                step(2 * r, 0, True)
                step(2 * r + 1, 1, True)
                return carry

            lax.fori_loop(1, NHALF // 2 - 1, body, 0)

            step(NHALF - 2, 0, True)
            step(NHALF - 1, 1, False)

        plsc.subcore_barrier()
        pltpu.sync_copy(acc_sh.at[pl.ds(r0, RPT)], out_hbm.at[c, pl.ds(r0, RPT)])

    return k(y, src, dst, zerosH)


def _tc_dinv(d0, d1):
    N, H = d0.shape

    def body(d0_ref, d1_ref, o_ref):
        deg = d0_ref[:, :1] + d1_ref[:, :1]
        o_ref[...] = lax.rsqrt(deg)

    return pl.pallas_call(
        body,
        grid=(_cdiv(N, _BN),),
        in_specs=[
            pl.BlockSpec((_BN, H), lambda i: (i, 0)),
            pl.BlockSpec((_BN, H), lambda i: (i, 0)),
        ],
        out_specs=pl.BlockSpec((_BN, 1), lambda i: (i, 0)),
        out_shape=jax.ShapeDtypeStruct((N, 1), jnp.float32),
    )(d0, d1)


def _tc_matmul_scale(x, W, dinv):
    N, D = x.shape
    H = W.shape[1]

    def body(x_ref, w_ref, dinv_ref, o_ref):
        y = jnp.dot(x_ref[...], w_ref[...], preferred_element_type=jnp.float32)
        o_ref[...] = dinv_ref[...] * y

    return pl.pallas_call(
        body,
        grid=(_cdiv(N, _BN),),
        in_specs=[
            pl.BlockSpec((_BN, D), lambda i: (i, 0)),
            pl.BlockSpec((D, H), lambda i: (0, 0)),
            pl.BlockSpec((_BN, 1), lambda i: (i, 0)),
        ],
        out_specs=pl.BlockSpec((_BN, H), lambda i: (i, 0)),
        out_shape=jax.ShapeDtypeStruct((N, H), jnp.float32),
    )(x, W, dinv)


def _tc_combine_matmul(p0, p1, dinv, b, W, bout, scale_out):
    N, D = p0.shape
    H = W.shape[1]

    def body(p0_ref, p1_ref, dinv_ref, b_ref, w_ref, bout_ref, o_ref):
        h = dinv_ref[...] * (p0_ref[...] + p1_ref[...]) + b_ref[...]
        h = jnp.maximum(h, 0.0)
        y = jnp.dot(h, w_ref[...], preferred_element_type=jnp.float32)
        if scale_out:
            y = dinv_ref[...] * y
        o_ref[...] = y + bout_ref[...]

    return pl.pallas_call(
        body,
        grid=(_cdiv(N, _BN),),
        in_specs=[
            pl.BlockSpec((_BN, D), lambda i: (i, 0)),
            pl.BlockSpec((_BN, D), lambda i: (i, 0)),
            pl.BlockSpec((_BN, 1), lambda i: (i, 0)),
            pl.BlockSpec((1, D), lambda i: (0, 0)),
            pl.BlockSpec((D, H), lambda i: (0, 0)),
            pl.BlockSpec((1, H), lambda i: (0, 0)),
        ],
        out_specs=pl.BlockSpec((_BN, H), lambda i: (i, 0)),
        out_shape=jax.ShapeDtypeStruct((N, H), jnp.float32),
    )(p0, p1, dinv, b, W, bout)


def kernel(x, edge_index, W1, b1, W2, b2, W3, b3, Wp, bp):
    N, D = x.shape
    E = edge_index.shape[1]
    NP = _cdiv(N, _NS * 8) * _NS * 8
    xp = jnp.pad(x, ((0, NP - N), (0, 0)))
    TPC = _cdiv(E, _NW * _CH * 4) * _CH * 4
    EP = _NW * TPC
    NCH = TPC // _CH
    src = jnp.pad(edge_index[0], (0, EP - E), constant_values=N).reshape(
        _NW, 2, NCH // 2, _CH)
    dst4 = jnp.pad(edge_index[1], (0, EP - E), constant_values=N).reshape(
        _NW, 2, NCH // 2, _CH)

    zerosH = jnp.zeros((NP, D), jnp.float32)
    onesH = jnp.ones((NP, D), jnp.float32)

    dpart = _sc_scatter_partials(onesH, dst4, dst4, zerosH)
    dinv = _tc_dinv(dpart[0], dpart[1])

    zH = jnp.zeros((1, W2.shape[1]), jnp.float32)
    y = _tc_matmul_scale(xp, W1, dinv)
    p = _sc_scatter_partials(y, src, dst4, zerosH)
    y = _tc_combine_matmul(p[0], p[1], dinv, b1.reshape(1, -1), W2, zH, True)
    p = _sc_scatter_partials(y, src, dst4, zerosH)
    y = _tc_combine_matmul(p[0], p[1], dinv, b2.reshape(1, -1), W3, zH, True)
    p = _sc_scatter_partials(y, src, dst4, zerosH)
    out = _tc_combine_matmul(p[0], p[1], dinv, b3.reshape(1, -1), Wp,
                             bp.reshape(1, -1), False)
    return out[:N]


# speedup vs baseline: 1.8845x; 1.8845x over previous
import functools

import jax
import jax.numpy as jnp
from jax import lax
from jax.experimental import pallas as pl
from jax.experimental.pallas import tpu as pltpu
from jax.experimental.pallas import tpu_sc as plsc

_NC = 2
_NS = 16
_NW = _NC * _NS
_CH = 80
_BN = 256


def _cdiv(a, b):
    return (a + b - 1) // b


def _sc_scatter_partials(y, src, dst, zerosH):
    N, H = y.shape
    (E,) = src.shape
    EPT = E // _NW
    n_chunks = EPT // _CH
    RPT = N // _NS
    mesh = plsc.VectorSubcoreMesh(core_axis_name="c", subcore_axis_name="s")

    @functools.partial(
        pl.kernel,
        out_type=jax.ShapeDtypeStruct((_NC, N, H), jnp.float32),
        mesh=mesh,
        scratch_types=[
            pltpu.VMEM((_CH,), jnp.int32),
            pltpu.VMEM((_CH,), jnp.int32),
            pltpu.VMEM((_CH, H), jnp.float32),
            pltpu.VMEM_SHARED((N, H), jnp.float32),
            pltpu.SemaphoreType.DMA,
        ],
    )
    def k(y_hbm, src_hbm, dst_hbm, zero_hbm, out_hbm, src_v, dst_v, rows_v, acc_sh, sem):
        c = lax.axis_index("c")
        s = lax.axis_index("s")
        wid = c * _NS + s
        r0 = s * RPT

        @pl.when(c == 0)
        def _():
            pltpu.sync_copy(y_hbm.at[pl.ds(r0, RPT)], acc_sh.at[pl.ds(r0, RPT)])

        @pl.when(c != 0)
        def _():
            pltpu.sync_copy(zero_hbm.at[pl.ds(r0, RPT)], acc_sh.at[pl.ds(r0, RPT)])

        plsc.subcore_barrier()
        base0 = wid * EPT

        def body(i, carry):
            base = base0 + i * _CH
            pltpu.sync_copy(src_hbm.at[pl.ds(base, _CH)], src_v)
            pltpu.sync_copy(dst_hbm.at[pl.ds(base, _CH)], dst_v)
            pltpu.async_copy(y_hbm.at[src_v], rows_v, sem).wait()
            pltpu.sync_copy(rows_v, acc_sh.at[dst_v], add=True)
            return carry

        lax.fori_loop(0, n_chunks, body, 0)
        plsc.subcore_barrier()
        pltpu.sync_copy(acc_sh.at[pl.ds(r0, RPT)], out_hbm.at[c, pl.ds(r0, RPT)])

    return k(y, src, dst, zerosH)


def _sc_deg_partials(dst, onesH, zerosH):
    N, H = onesH.shape
    (E,) = dst.shape
    EPT = E // _NW
    n_chunks = EPT // _CH
    RPT = N // _NS
    mesh = plsc.VectorSubcoreMesh(core_axis_name="c", subcore_axis_name="s")

    @functools.partial(
        pl.kernel,
        out_type=jax.ShapeDtypeStruct((_NC, N, H), jnp.float32),
        mesh=mesh,
        scratch_types=[
            pltpu.VMEM((_CH,), jnp.int32),
            pltpu.VMEM((_CH, H), jnp.float32),
            pltpu.VMEM_SHARED((N, H), jnp.float32),
        ],
    )
    def k(dst_hbm, ones_hbm, zero_hbm, out_hbm, dst_v, ones_v, acc_sh):
        c = lax.axis_index("c")
        s = lax.axis_index("s")
        wid = c * _NS + s
        r0 = s * RPT

        @pl.when(c == 0)
        def _():
            pltpu.sync_copy(ones_hbm.at[pl.ds(r0, RPT)], acc_sh.at[pl.ds(r0, RPT)])

        @pl.when(c != 0)
        def _():
            pltpu.sync_copy(zero_hbm.at[pl.ds(r0, RPT)], acc_sh.at[pl.ds(r0, RPT)])

        pltpu.sync_copy(ones_hbm.at[pl.ds(0, _CH)], ones_v)
        plsc.subcore_barrier()
        base0 = wid * EPT

        def body(i, carry):
            pltpu.sync_copy(dst_hbm.at[pl.ds(base0 + i * _CH, _CH)], dst_v)
            pltpu.sync_copy(ones_v, acc_sh.at[dst_v], add=True)
            return carry

        lax.fori_loop(0, n_chunks, body, 0)
        plsc.subcore_barrier()
        pltpu.sync_copy(acc_sh.at[pl.ds(r0, RPT)], out_hbm.at[c, pl.ds(r0, RPT)])

    return k(dst, onesH, zerosH)


def _tc_dinv(d0, d1):
    N, H = d0.shape

    def body(d0_ref, d1_ref, o_ref):
        deg = d0_ref[:, :1] + d1_ref[:, :1]
        o_ref[...] = lax.rsqrt(deg)

    return pl.pallas_call(
        body,
        grid=(_cdiv(N, _BN),),
        in_specs=[
            pl.BlockSpec((_BN, H), lambda i: (i, 0)),
            pl.BlockSpec((_BN, H), lambda i: (i, 0)),
        ],
        out_specs=pl.BlockSpec((_BN, 1), lambda i: (i, 0)),
        out_shape=jax.ShapeDtypeStruct((N, 1), jnp.float32),
    )(d0, d1)


def _tc_matmul_scale(x, W, dinv):
    N, D = x.shape
    H = W.shape[1]

    def body(x_ref, w_ref, dinv_ref, o_ref):
        y = jnp.dot(x_ref[...], w_ref[...], preferred_element_type=jnp.float32)
        o_ref[...] = dinv_ref[...] * y

    return pl.pallas_call(
        body,
        grid=(_cdiv(N, _BN),),
        in_specs=[
            pl.BlockSpec((_BN, D), lambda i: (i, 0)),
            pl.BlockSpec((D, H), lambda i: (0, 0)),
            pl.BlockSpec((_BN, 1), lambda i: (i, 0)),
        ],
        out_specs=pl.BlockSpec((_BN, H), lambda i: (i, 0)),
        out_shape=jax.ShapeDtypeStruct((N, H), jnp.float32),
    )(x, W, dinv)


def _tc_combine_matmul(p0, p1, dinv, b, W, bout, scale_out):
    N, D = p0.shape
    H = W.shape[1]

    def body(p0_ref, p1_ref, dinv_ref, b_ref, w_ref, bout_ref, o_ref):
        h = dinv_ref[...] * (p0_ref[...] + p1_ref[...]) + b_ref[...]
        h = jnp.maximum(h, 0.0)
        y = jnp.dot(h, w_ref[...], preferred_element_type=jnp.float32)
        if scale_out:
            y = dinv_ref[...] * y
        o_ref[...] = y + bout_ref[...]

    return pl.pallas_call(
        body,
        grid=(_cdiv(N, _BN),),
        in_specs=[
            pl.BlockSpec((_BN, D), lambda i: (i, 0)),
            pl.BlockSpec((_BN, D), lambda i: (i, 0)),
            pl.BlockSpec((_BN, 1), lambda i: (i, 0)),
            pl.BlockSpec((1, D), lambda i: (0, 0)),
            pl.BlockSpec((D, H), lambda i: (0, 0)),
            pl.BlockSpec((1, H), lambda i: (0, 0)),
        ],
        out_specs=pl.BlockSpec((_BN, H), lambda i: (i, 0)),
        out_shape=jax.ShapeDtypeStruct((N, H), jnp.float32),
    )(p0, p1, dinv, b, W, bout)


def kernel(x, edge_index, W1, b1, W2, b2, W3, b3, Wp, bp):
    N, D = x.shape
    NP = _cdiv(N, _NS * 8) * _NS * 8
    xp = jnp.pad(x, ((0, NP - N), (0, 0)))
    src = edge_index[0]
    dst = edge_index[1]

    zerosH = jnp.zeros((NP, D), jnp.float32)
    onesH = jnp.ones((NP, D), jnp.float32)

    dpart = _sc_deg_partials(dst, onesH, zerosH)
    dinv = _tc_dinv(dpart[0], dpart[1])

    zH = jnp.zeros((1, W2.shape[1]), jnp.float32)
    y = _tc_matmul_scale(xp, W1, dinv)
    p = _sc_scatter_partials(y, src, dst, zerosH)
    y = _tc_combine_matmul(p[0], p[1], dinv, b1.reshape(1, -1), W2, zH, True)
    p = _sc_scatter_partials(y, src, dst, zerosH)
    y = _tc_combine_matmul(p[0], p[1], dinv, b2.reshape(1, -1), W3, zH, True)
    p = _sc_scatter_partials(y, src, dst, zerosH)
    out = _tc_combine_matmul(p[0], p[1], dinv, b3.reshape(1, -1), Wp,
                             bp.reshape(1, -1), False)
    return out[:N]


# speedup vs baseline: 2.4943x; 1.3236x over previous
import functools

import jax
import jax.numpy as jnp
from jax import lax
from jax.experimental import pallas as pl
from jax.experimental.pallas import tpu as pltpu
from jax.experimental.pallas import tpu_sc as plsc

_NC = 2
_NS = 16
_NW = _NC * _NS
_CH = 80
_BN = 256


def _cdiv(a, b):
    return (a + b - 1) // b


def _sc_scatter_partials(y, src, dst, zerosH):
    N, H = y.shape
    (E,) = src.shape
    EPT = E // _NW
    n_chunks = EPT // _CH
    RPT = N // _NS
    mesh = plsc.VectorSubcoreMesh(core_axis_name="c", subcore_axis_name="s")

    @functools.partial(
        pl.kernel,
        out_type=jax.ShapeDtypeStruct((_NC, N, H), jnp.float32),
        mesh=mesh,
        scratch_types=[
            pltpu.VMEM((_CH,), jnp.int32),
            pltpu.VMEM((_CH,), jnp.int32),
            pltpu.VMEM((_CH,), jnp.int32),
            pltpu.VMEM((_CH,), jnp.int32),
            pltpu.VMEM((_CH, H), jnp.float32),
            pltpu.VMEM_SHARED((N, H), jnp.float32),
            pltpu.SemaphoreType.DMA,
            pltpu.SemaphoreType.DMA,
        ],
    )
    def k(y_hbm, src_hbm, dst_hbm, zero_hbm, out_hbm, sv0, sv1, dv0, dv1,
          rows_v, acc_sh, gsem, isem):
        srcs = (sv0, sv1)
        dsts = (dv0, dv1)
        c = lax.axis_index("c")
        s = lax.axis_index("s")
        wid = c * _NS + s
        r0 = s * RPT

        @pl.when(c == 0)
        def _():
            pltpu.sync_copy(y_hbm.at[pl.ds(r0, RPT)], acc_sh.at[pl.ds(r0, RPT)])

        @pl.when(c != 0)
        def _():
            pltpu.sync_copy(zero_hbm.at[pl.ds(r0, RPT)], acc_sh.at[pl.ds(r0, RPT)])

        plsc.subcore_barrier()
        base0 = wid * EPT

        def fire_i(i, b):
            base = base0 + i * _CH
            pltpu.async_copy(src_hbm.at[pl.ds(base, _CH)], srcs[b], isem)
            pltpu.async_copy(dst_hbm.at[pl.ds(base, _CH)], dsts[b], isem)

        def wait_i(b):
            pltpu.make_async_copy(src_hbm.at[pl.ds(0, _CH)], srcs[b], isem).wait()
            pltpu.make_async_copy(dst_hbm.at[pl.ds(0, _CH)], dsts[b], isem).wait()

        def step(i, b, fire_next):
            wait_i(b)
            if fire_next:
                fire_i(i + 1, 1 - b)
            pltpu.async_copy(y_hbm.at[srcs[b]], rows_v, gsem).wait()
            pltpu.sync_copy(rows_v, acc_sh.at[dsts[b]], add=True)

        rounds = (n_chunks - 3) // 2
        fire_i(0, 0)
        step(0, 0, True)

        def body(r, carry):
            step(2 * r + 1, 1, True)
            step(2 * r + 2, 0, True)
            return carry

        lax.fori_loop(0, rounds, body, 0)
        step(n_chunks - 2, 1, True)
        step(n_chunks - 1, 0, False)

        plsc.subcore_barrier()
        pltpu.sync_copy(acc_sh.at[pl.ds(r0, RPT)], out_hbm.at[c, pl.ds(r0, RPT)])

    return k(y, src, dst, zerosH)


def _sc_deg_partials(dst, onesH, zerosH):
    N, H = onesH.shape
    (E,) = dst.shape
    EPT = E // _NW
    n_chunks = EPT // _CH
    RPT = N // _NS
    mesh = plsc.VectorSubcoreMesh(core_axis_name="c", subcore_axis_name="s")

    @functools.partial(
        pl.kernel,
        out_type=jax.ShapeDtypeStruct((_NC, N, H), jnp.float32),
        mesh=mesh,
        scratch_types=[
            pltpu.VMEM((_CH,), jnp.int32),
            pltpu.VMEM((_CH, H), jnp.float32),
            pltpu.VMEM_SHARED((N, H), jnp.float32),
        ],
    )
    def k(dst_hbm, ones_hbm, zero_hbm, out_hbm, dst_v, ones_v, acc_sh):
        c = lax.axis_index("c")
        s = lax.axis_index("s")
        wid = c * _NS + s
        r0 = s * RPT

        @pl.when(c == 0)
        def _():
            pltpu.sync_copy(ones_hbm.at[pl.ds(r0, RPT)], acc_sh.at[pl.ds(r0, RPT)])

        @pl.when(c != 0)
        def _():
            pltpu.sync_copy(zero_hbm.at[pl.ds(r0, RPT)], acc_sh.at[pl.ds(r0, RPT)])

        pltpu.sync_copy(ones_hbm.at[pl.ds(0, _CH)], ones_v)
        plsc.subcore_barrier()
        base0 = wid * EPT

        def body(i, carry):
            pltpu.sync_copy(dst_hbm.at[pl.ds(base0 + i * _CH, _CH)], dst_v)
            pltpu.sync_copy(ones_v, acc_sh.at[dst_v], add=True)
            return carry

        lax.fori_loop(0, n_chunks, body, 0)
        plsc.subcore_barrier()
        pltpu.sync_copy(acc_sh.at[pl.ds(r0, RPT)], out_hbm.at[c, pl.ds(r0, RPT)])

    return k(dst, onesH, zerosH)


def _tc_dinv(d0, d1):
    N, H = d0.shape

    def body(d0_ref, d1_ref, o_ref):
        deg = d0_ref[:, :1] + d1_ref[:, :1]
        o_ref[...] = lax.rsqrt(deg)

    return pl.pallas_call(
        body,
        grid=(_cdiv(N, _BN),),
        in_specs=[
            pl.BlockSpec((_BN, H), lambda i: (i, 0)),
            pl.BlockSpec((_BN, H), lambda i: (i, 0)),
        ],
        out_specs=pl.BlockSpec((_BN, 1), lambda i: (i, 0)),
        out_shape=jax.ShapeDtypeStruct((N, 1), jnp.float32),
    )(d0, d1)


def _tc_matmul_scale(x, W, dinv):
    N, D = x.shape
    H = W.shape[1]

    def body(x_ref, w_ref, dinv_ref, o_ref):
        y = jnp.dot(x_ref[...], w_ref[...], preferred_element_type=jnp.float32)
        o_ref[...] = dinv_ref[...] * y

    return pl.pallas_call(
        body,
        grid=(_cdiv(N, _BN),),
        in_specs=[
            pl.BlockSpec((_BN, D), lambda i: (i, 0)),
            pl.BlockSpec((D, H), lambda i: (0, 0)),
            pl.BlockSpec((_BN, 1), lambda i: (i, 0)),
        ],
        out_specs=pl.BlockSpec((_BN, H), lambda i: (i, 0)),
        out_shape=jax.ShapeDtypeStruct((N, H), jnp.float32),
    )(x, W, dinv)


def _tc_combine_matmul(p0, p1, dinv, b, W, bout, scale_out):
    N, D = p0.shape
    H = W.shape[1]

    def body(p0_ref, p1_ref, dinv_ref, b_ref, w_ref, bout_ref, o_ref):
        h = dinv_ref[...] * (p0_ref[...] + p1_ref[...]) + b_ref[...]
        h = jnp.maximum(h, 0.0)
        y = jnp.dot(h, w_ref[...], preferred_element_type=jnp.float32)
        if scale_out:
            y = dinv_ref[...] * y
        o_ref[...] = y + bout_ref[...]

    return pl.pallas_call(
        body,
        grid=(_cdiv(N, _BN),),
        in_specs=[
            pl.BlockSpec((_BN, D), lambda i: (i, 0)),
            pl.BlockSpec((_BN, D), lambda i: (i, 0)),
            pl.BlockSpec((_BN, 1), lambda i: (i, 0)),
            pl.BlockSpec((1, D), lambda i: (0, 0)),
            pl.BlockSpec((D, H), lambda i: (0, 0)),
            pl.BlockSpec((1, H), lambda i: (0, 0)),
        ],
        out_specs=pl.BlockSpec((_BN, H), lambda i: (i, 0)),
        out_shape=jax.ShapeDtypeStruct((N, H), jnp.float32),
    )(p0, p1, dinv, b, W, bout)


def kernel(x, edge_index, W1, b1, W2, b2, W3, b3, Wp, bp):
    N, D = x.shape
    NP = _cdiv(N, _NS * 8) * _NS * 8
    xp = jnp.pad(x, ((0, NP - N), (0, 0)))
    src = edge_index[0]
    dst = edge_index[1]

    zerosH = jnp.zeros((NP, D), jnp.float32)
    onesH = jnp.ones((NP, D), jnp.float32)

    dpart = _sc_deg_partials(dst, onesH, zerosH)
    dinv = _tc_dinv(dpart[0], dpart[1])

    zH = jnp.zeros((1, W2.shape[1]), jnp.float32)
    y = _tc_matmul_scale(xp, W1, dinv)
    p = _sc_scatter_partials(y, src, dst, zerosH)
    y = _tc_combine_matmul(p[0], p[1], dinv, b1.reshape(1, -1), W2, zH, True)
    p = _sc_scatter_partials(y, src, dst, zerosH)
    y = _tc_combine_matmul(p[0], p[1], dinv, b2.reshape(1, -1), W3, zH, True)
    p = _sc_scatter_partials(y, src, dst, zerosH)
    out = _tc_combine_matmul(p[0], p[1], dinv, b3.reshape(1, -1), Wp,
                             bp.reshape(1, -1), False)
    return out[:N]


# speedup vs baseline: 2.4963x; 1.0008x over previous
import functools

import jax
import jax.numpy as jnp
from jax import lax
from jax.experimental import pallas as pl
from jax.experimental.pallas import tpu as pltpu
from jax.experimental.pallas import tpu_sc as plsc

_NC = 2
_NS = 16
_NW = _NC * _NS
_CH = 80
_BN = 256


def _cdiv(a, b):
    return (a + b - 1) // b


def _sc_scatter_partials(y, src, dst, zerosH):
    N, H = y.shape
    (E,) = src.shape
    EPT = E // _NW
    n_chunks = EPT // _CH
    RPT = N // _NS
    mesh = plsc.VectorSubcoreMesh(core_axis_name="c", subcore_axis_name="s")

    @functools.partial(
        pl.kernel,
        out_type=jax.ShapeDtypeStruct((_NC, N, H), jnp.float32),
        mesh=mesh,
        scratch_types=[
            pltpu.VMEM((_CH,), jnp.int32),
            pltpu.VMEM((_CH,), jnp.int32),
            pltpu.VMEM((_CH,), jnp.int32),
            pltpu.VMEM((_CH,), jnp.int32),
            pltpu.VMEM((_CH, H), jnp.float32),
            pltpu.VMEM_SHARED((N, H), jnp.float32),
            pltpu.SemaphoreType.DMA,
            pltpu.SemaphoreType.DMA,
        ],
    )
    def k(y_hbm, src_hbm, dst_hbm, zero_hbm, out_hbm, sv0, sv1, dv0, dv1,
          rows_v, acc_sh, gsem, isem):
        srcs = (sv0, sv1)
        dsts = (dv0, dv1)
        c = lax.axis_index("c")
        s = lax.axis_index("s")
        wid = c * _NS + s
        r0 = s * RPT

        @pl.when(c == 0)
        def _():
            pltpu.sync_copy(y_hbm.at[pl.ds(r0, RPT)], acc_sh.at[pl.ds(r0, RPT)])

        @pl.when(c != 0)
        def _():
            pltpu.sync_copy(zero_hbm.at[pl.ds(r0, RPT)], acc_sh.at[pl.ds(r0, RPT)])

        plsc.subcore_barrier()
        base0 = wid * EPT

        def fire_i(i, b):
            base = base0 + i * _CH
            pltpu.async_copy(src_hbm.at[pl.ds(base, _CH)], srcs[b], isem)
            pltpu.async_copy(dst_hbm.at[pl.ds(base, _CH)], dsts[b], isem)

        def wait_i(b):
            pltpu.make_async_copy(src_hbm.at[pl.ds(0, _CH)], srcs[b], isem).wait()
            pltpu.make_async_copy(dst_hbm.at[pl.ds(0, _CH)], dsts[b], isem).wait()

        def step(i, b, fire_next):
            wait_i(b)
            if fire_next:
                fire_i(i + 1, 1 - b)
            pltpu.async_copy(y_hbm.at[srcs[b]], rows_v, gsem).wait()
            pltpu.sync_copy(rows_v, acc_sh.at[dsts[b]], add=True)

        rounds = (n_chunks - 3) // 2
        fire_i(0, 0)
        step(0, 0, True)

        def body(r, carry):
            step(2 * r + 1, 1, True)
            step(2 * r + 2, 0, True)
            return carry

        lax.fori_loop(0, rounds, body, 0)
        step(n_chunks - 2, 1, True)
        step(n_chunks - 1, 0, False)

        plsc.subcore_barrier()
        pltpu.sync_copy(acc_sh.at[pl.ds(r0, RPT)], out_hbm.at[c, pl.ds(r0, RPT)])

    return k(y, src, dst, zerosH)


def _sc_deg_partials(dst, onesH, zerosH):
    N, H = onesH.shape
    (E,) = dst.shape
    EPT = E // _NW
    n_chunks = EPT // _CH
    RPT = N // _NS
    mesh = plsc.VectorSubcoreMesh(core_axis_name="c", subcore_axis_name="s")

    @functools.partial(
        pl.kernel,
        out_type=jax.ShapeDtypeStruct((_NC, N, H), jnp.float32),
        mesh=mesh,
        scratch_types=[
            pltpu.VMEM((_CH,), jnp.int32),
            pltpu.VMEM((_CH, H), jnp.float32),
            pltpu.VMEM_SHARED((N, H), jnp.float32),
            pltpu.SemaphoreType.DMA,
        ],
    )
    def k(dst_hbm, ones_hbm, zero_hbm, out_hbm, dst_v, ones_v, acc_sh, isem):
        c = lax.axis_index("c")
        s = lax.axis_index("s")
        wid = c * _NS + s
        r0 = s * RPT

        @pl.when(c == 0)
        def _():
            pltpu.sync_copy(ones_hbm.at[pl.ds(r0, RPT)], acc_sh.at[pl.ds(r0, RPT)])

        @pl.when(c != 0)
        def _():
            pltpu.sync_copy(zero_hbm.at[pl.ds(r0, RPT)], acc_sh.at[pl.ds(r0, RPT)])

        pltpu.sync_copy(ones_hbm.at[pl.ds(0, _CH)], ones_v)
        plsc.subcore_barrier()
        base0 = wid * EPT

        def body(i, carry):
            pltpu.sync_copy(dst_hbm.at[pl.ds(base0 + i * _CH, _CH)], dst_v)
            pltpu.sync_copy(ones_v, acc_sh.at[dst_v], add=True)
            return carry

        lax.fori_loop(0, n_chunks, body, 0)
        plsc.subcore_barrier()
        pltpu.sync_copy(acc_sh.at[pl.ds(r0, RPT)], out_hbm.at[c, pl.ds(r0, RPT)])

    return k(dst, onesH, zerosH)


def _tc_dinv(d0, d1):
    N, H = d0.shape

    def body(d0_ref, d1_ref, o_ref):
        deg = d0_ref[:, :1] + d1_ref[:, :1]
        o_ref[...] = lax.rsqrt(deg)

    return pl.pallas_call(
        body,
        grid=(_cdiv(N, _BN),),
        in_specs=[
            pl.BlockSpec((_BN, H), lambda i: (i, 0)),
            pl.BlockSpec((_BN, H), lambda i: (i, 0)),
        ],
        out_specs=pl.BlockSpec((_BN, 1), lambda i: (i, 0)),
        out_shape=jax.ShapeDtypeStruct((N, 1), jnp.float32),
    )(d0, d1)


def _tc_matmul_scale(x, W, dinv):
    N, D = x.shape
    H = W.shape[1]

    def body(x_ref, w_ref, dinv_ref, o_ref):
        y = jnp.dot(x_ref[...], w_ref[...], preferred_element_type=jnp.float32)
        o_ref[...] = dinv_ref[...] * y

    return pl.pallas_call(
        body,
        grid=(_cdiv(N, _BN),),
        in_specs=[
            pl.BlockSpec((_BN, D), lambda i: (i, 0)),
            pl.BlockSpec((D, H), lambda i: (0, 0)),
            pl.BlockSpec((_BN, 1), lambda i: (i, 0)),
        ],
        out_specs=pl.BlockSpec((_BN, H), lambda i: (i, 0)),
        out_shape=jax.ShapeDtypeStruct((N, H), jnp.float32),
    )(x, W, dinv)


def _tc_combine_matmul(p0, p1, dinv, b, W, bout, scale_out):
    N, D = p0.shape
    H = W.shape[1]

    def body(p0_ref, p1_ref, dinv_ref, b_ref, w_ref, bout_ref, o_ref):
        h = dinv_ref[...] * (p0_ref[...] + p1_ref[...]) + b_ref[...]
        h = jnp.maximum(h, 0.0)
        y = jnp.dot(h, w_ref[...], preferred_element_type=jnp.float32)
        if scale_out:
            y = dinv_ref[...] * y
        o_ref[...] = y + bout_ref[...]

    return pl.pallas_call(
        body,
        grid=(_cdiv(N, _BN),),
        in_specs=[
            pl.BlockSpec((_BN, D), lambda i: (i, 0)),
            pl.BlockSpec((_BN, D), lambda i: (i, 0)),
            pl.BlockSpec((_BN, 1), lambda i: (i, 0)),
            pl.BlockSpec((1, D), lambda i: (0, 0)),
            pl.BlockSpec((D, H), lambda i: (0, 0)),
            pl.BlockSpec((1, H), lambda i: (0, 0)),
        ],
        out_specs=pl.BlockSpec((_BN, H), lambda i: (i, 0)),
        out_shape=jax.ShapeDtypeStruct((N, H), jnp.float32),
    )(p0, p1, dinv, b, W, bout)


def kernel(x, edge_index, W1, b1, W2, b2, W3, b3, Wp, bp):
    N, D = x.shape
    NP = _cdiv(N, _NS * 8) * _NS * 8
    xp = jnp.pad(x, ((0, NP - N), (0, 0)))
    src = edge_index[0]
    dst = edge_index[1]

    zerosH = jnp.zeros((NP, D), jnp.float32)
    onesH = jnp.ones((NP, D), jnp.float32)

    dpart = _sc_deg_partials(dst, onesH, zerosH)
    dinv = _tc_dinv(dpart[0], dpart[1])

    zH = jnp.zeros((1, W2.shape[1]), jnp.float32)
    y = _tc_matmul_scale(xp, W1, dinv)
    p = _sc_scatter_partials(y, src, dst, zerosH)
    y = _tc_combine_matmul(p[0], p[1], dinv, b1.reshape(1, -1), W2, zH, True)
    p = _sc_scatter_partials(y, src, dst, zerosH)
    y = _tc_combine_matmul(p[0], p[1], dinv, b2.reshape(1, -1), W3, zH, True)
    p = _sc_scatter_partials(y, src, dst, zerosH)
    out = _tc_combine_matmul(p[0], p[1], dinv, b3.reshape(1, -1), Wp,
                             bp.reshape(1, -1), False)
    return out[:N]
